# baseline (speedup 1.0000x reference)
import jax
import jax.numpy as jnp
from jax.experimental import pallas as pl


def _sx_kernel(x_ref, w_ref, o_ref):
    o_ref[...] = jnp.dot(x_ref[...], w_ref[...],
                         preferred_element_type=jnp.float32)


def _pass1_kernel(adj_ref, s1_ref, b1_ref, w2_ref, s2_ref):
    h = jnp.dot(adj_ref[...], s1_ref[...],
                preferred_element_type=jnp.float32)
    h = jnp.maximum(h + b1_ref[...], 0.0)
    s2_ref[...] = jnp.dot(h, w2_ref[...],
                          preferred_element_type=jnp.float32)


def _pass2_kernel(adj_ref, s2_ref, b2_ref, wfc_ref, bfc_ref, o_ref):
    h = jnp.dot(adj_ref[...], s2_ref[...],
                preferred_element_type=jnp.float32)
    h = h + b2_ref[...]
    logits = jax.lax.dot_general(
        h, wfc_ref[...], (((1,), (1,)), ((), ())),
        preferred_element_type=jnp.float32) + bfc_ref[...]
    m = jnp.max(logits, axis=1, keepdims=True)
    lse = jnp.log(jnp.sum(jnp.exp(logits - m), axis=1, keepdims=True))
    o_ref[...] = (logits - m) - lse


def kernel(x, adj, W1, b1, W2, b2, Wfc, bfc):
    n, nf = x.shape
    nh = W1.shape[1]
    nc = Wfc.shape[0]
    b1r = b1.reshape(1, nh)
    b2r = b2.reshape(1, nh)
    bfcr = bfc.reshape(1, nc)

    s1 = pl.pallas_call(
        _sx_kernel,
        out_shape=jax.ShapeDtypeStruct((n, nh), jnp.float32),
    )(x, W1)

    rb = 200
    grid = (n // rb,)

    s2 = pl.pallas_call(
        _pass1_kernel,
        grid=grid,
        in_specs=[
            pl.BlockSpec((rb, n), lambda i: (i, 0)),
            pl.BlockSpec((n, nh), lambda i: (0, 0)),
            pl.BlockSpec((1, nh), lambda i: (0, 0)),
            pl.BlockSpec((nh, nh), lambda i: (0, 0)),
        ],
        out_specs=pl.BlockSpec((rb, nh), lambda i: (i, 0)),
        out_shape=jax.ShapeDtypeStruct((n, nh), jnp.float32),
    )(adj, s1, b1r, W2)

    out = pl.pallas_call(
        _pass2_kernel,
        grid=grid,
        in_specs=[
            pl.BlockSpec((rb, n), lambda i: (i, 0)),
            pl.BlockSpec((n, nh), lambda i: (0, 0)),
            pl.BlockSpec((1, nh), lambda i: (0, 0)),
            pl.BlockSpec((nc, nh), lambda i: (0, 0)),
            pl.BlockSpec((1, nc), lambda i: (0, 0)),
        ],
        out_specs=pl.BlockSpec((rb, nc), lambda i: (i, 0)),
        out_shape=jax.ShapeDtypeStruct((n, nc), jnp.float32),
    )(adj, s2, b2r, Wfc, bfcr)

    return out


# speedup vs baseline: 1.0006x; 1.0006x over previous
import jax
import jax.numpy as jnp
from jax.experimental import pallas as pl
from jax.experimental.pallas import tpu as pltpu


def _sx_kernel(x_ref, w_ref, o_ref):
    o_ref[...] = jnp.dot(x_ref[...], w_ref[...],
                         preferred_element_type=jnp.float32)


def _pass1_kernel(adj_ref, s1_ref, b1_ref, w2_ref, s2_ref):
    h = jnp.dot(adj_ref[...], s1_ref[...],
                preferred_element_type=jnp.float32)
    h = jnp.maximum(h + b1_ref[...], 0.0)
    s2_ref[...] = jnp.dot(h, w2_ref[...],
                          preferred_element_type=jnp.float32)


def _pass2_kernel(adj_ref, s2_ref, b2_ref, wfc_ref, bfc_ref, o_ref):
    h = jnp.dot(adj_ref[...], s2_ref[...],
                preferred_element_type=jnp.float32)
    h = h + b2_ref[...]
    logits = jax.lax.dot_general(
        h, wfc_ref[...], (((1,), (1,)), ((), ())),
        preferred_element_type=jnp.float32) + bfc_ref[...]
    m = jnp.max(logits, axis=1, keepdims=True)
    lse = jnp.log(jnp.sum(jnp.exp(logits - m), axis=1, keepdims=True))
    o_ref[...] = (logits - m) - lse


def kernel(x, adj, W1, b1, W2, b2, Wfc, bfc):
    n, nf = x.shape
    nh = W1.shape[1]
    nc = Wfc.shape[0]
    b1r = b1.reshape(1, nh)
    b2r = b2.reshape(1, nh)
    bfcr = bfc.reshape(1, nc)

    s1 = pl.pallas_call(
        _sx_kernel,
        out_shape=jax.ShapeDtypeStruct((n, nh), jnp.float32),
    )(x, W1)

    rb = 200
    grid = (n // rb,)

    s2 = pl.pallas_call(
        _pass1_kernel,
        grid=grid,
        in_specs=[
            pl.BlockSpec((rb, n), lambda i: (i, 0)),
            pl.BlockSpec((n, nh), lambda i: (0, 0)),
            pl.BlockSpec((1, nh), lambda i: (0, 0)),
            pl.BlockSpec((nh, nh), lambda i: (0, 0)),
        ],
        out_specs=pl.BlockSpec((rb, nh), lambda i: (i, 0)),
        out_shape=jax.ShapeDtypeStruct((n, nh), jnp.float32),
        compiler_params=pltpu.CompilerParams(
            dimension_semantics=("parallel",)),
    )(adj, s1, b1r, W2)

    out = pl.pallas_call(
        _pass2_kernel,
        grid=grid,
        in_specs=[
            pl.BlockSpec((rb, n), lambda i: (i, 0)),
            pl.BlockSpec((n, nh), lambda i: (0, 0)),
            pl.BlockSpec((1, nh), lambda i: (0, 0)),
            pl.BlockSpec((nc, nh), lambda i: (0, 0)),
            pl.BlockSpec((1, nc), lambda i: (0, 0)),
        ],
        out_specs=pl.BlockSpec((rb, nc), lambda i: (i, 0)),
        out_shape=jax.ShapeDtypeStruct((n, nc), jnp.float32),
        compiler_params=pltpu.CompilerParams(
            dimension_semantics=("parallel",)),
    )(adj, s2, b2r, Wfc, bfcr)

    return out
